# baseline (speedup 1.0000x reference)
import functools

import jax
import jax.numpy as jnp
from jax.experimental import pallas as pl

NC = 81
BIGF = 3.0e38
BIGI = 2 ** 30
HIGHEST = jax.lax.Precision.HIGHEST


def _pass_a(bn, ca_ref, box_ref, m_ref, am_ref, pom_ref, poa_ref):
    kb = pl.program_id(1)
    ca = ca_ref[...]
    ax1 = ca[0:1, :] - ca[2:3, :] / 2.0
    ay1 = ca[1:2, :] - ca[3:4, :] / 2.0
    ax2 = ca[0:1, :] + ca[2:3, :] / 2.0
    ay2 = ca[1:2, :] + ca[3:4, :] / 2.0
    b = box_ref[0]
    bx1, by1, bx2, by2 = b[:, 0:1], b[:, 1:2], b[:, 2:3], b[:, 3:4]
    iw = jnp.clip(jnp.minimum(ax2, bx2) - jnp.maximum(ax1, bx1), 0.0)
    ih = jnp.clip(jnp.minimum(ay2, by2) - jnp.maximum(ay1, by1), 0.0)
    inter = iw * ih
    aarea = (ax2 - ax1) * (ay2 - ay1)
    barea = (bx2 - bx1) * (by2 - by1)
    iou = inter / (aarea + barea - inter)

    m = jnp.max(iou, axis=0, keepdims=True)
    iids = jax.lax.broadcasted_iota(jnp.int32, iou.shape, 0)
    am = jnp.min(jnp.where(iou == m, iids, BIGI), axis=0, keepdims=True)
    m_ref[...] = m.reshape(1, 1, bn)
    am_ref[...] = am.reshape(1, 1, bn)

    pm = jnp.max(iou, axis=1, keepdims=True)
    gids = jax.lax.broadcasted_iota(jnp.int32, iou.shape, 1) + kb * bn
    pa = jnp.min(jnp.where(iou == pm, gids, BIGI), axis=1, keepdims=True)
    pom_ref[...] = pm.reshape(1, 1, -1, 1)
    poa_ref[...] = pa.reshape(1, 1, -1, 1)


def _pass_b(bn, nblocks, ca_ref, bt_ref, lab_ref, m_ref, am_ref, pom_ref,
            poa_ref, cls_ref, loc_ref, aid_ref):
    kb = pl.program_id(1)
    n_obj = bt_ref.shape[-1]

    best = pom_ref[0, 0]
    for k2 in range(1, nblocks):
        best = jnp.maximum(best, pom_ref[0, k2])
    gidx = jnp.full_like(poa_ref[0, 0], BIGI)
    for k2 in range(nblocks):
        gidx = jnp.minimum(
            gidx, jnp.where(pom_ref[0, k2] == best, poa_ref[0, k2], BIGI))
    any_pos = jnp.max(best) >= 0.5

    m = m_ref[0]
    am = am_ref[0]
    sel = (jax.lax.broadcasted_iota(jnp.int32, (n_obj, bn), 0)
           == am).astype(jnp.float32)

    bt = bt_ref[0]
    lab5 = jnp.concatenate([bt, lab_ref[0]], axis=0)
    bl = jax.lax.dot_general(lab5, sel, (((1,), (0,)), ((), ())),
                             precision=HIGHEST)

    gi = jax.lax.broadcasted_iota(jnp.int32, (n_obj, bn), 1) + kb * bn
    fb = jnp.any(gi == gidx, axis=0, keepdims=True)
    pos = (any_pos & (m >= 0.5)) | (jnp.logical_not(any_pos) & fb)
    neg = m < 0.4
    posf = pos.astype(jnp.float32)
    ignf = (1.0 - posf) * (1.0 - neg.astype(jnp.float32))
    aid_ref[...] = (posf - ignf).reshape(1, 1, bn)

    onehot = (jax.lax.broadcasted_iota(jnp.int32, (n_obj, 128), 1)
              == lab_ref[0].astype(jnp.int32).reshape(n_obj, 1) + 1
              ).astype(jnp.float32)
    ign_row = jnp.where(
        jax.lax.broadcasted_iota(jnp.int32, (1, 128), 1) < NC, -1.0, 0.0)
    mmat = jnp.concatenate([onehot, ign_row], axis=0)
    smat = jnp.concatenate([sel * posf, ignf], axis=0)
    cls = jax.lax.dot_general(smat, mmat, (((0,), (0,)), ((), ())),
                              precision=jax.lax.Precision.DEFAULT)
    cls_ref[...] = cls.reshape(1, bn, 128)

    ca = ca_ref[...]
    x1, y1, x2, y2 = bl[0:1, :], bl[1:2, :], bl[2:3, :], bl[3:4, :]
    cx = (x1 + x2) / 2.0
    cy = (y1 + y2) / 2.0
    w = x2 - x1
    h = y2 - y1
    gcx = (cx - ca[0:1, :]) / (ca[2:3, :] / 10.0)
    gcy = (cy - ca[1:2, :]) / (ca[3:4, :] / 10.0)
    gw = jnp.log(w / ca[2:3, :]) * 5.0
    gh = jnp.log(h / ca[3:4, :]) * 5.0
    loc_t = jnp.concatenate([gcx, gcy, gw, gh], axis=0)
    loc_ref[...] = loc_t.reshape(1, 4, bn)


@jax.jit
def kernel(gt_boxes, gt_labels, center_anchor):
    B, n_obj = gt_labels.shape
    np_real = center_anchor.shape[0]
    BN = 10240
    nblocks = -(-np_real // BN)
    np_pad = nblocks * BN

    pad = jnp.broadcast_to(jnp.array([2.0, 2.0, 0.01, 0.01], jnp.float32),
                           (np_pad - np_real, 4))
    ca_t = jnp.transpose(jnp.concatenate([center_anchor, pad], axis=0))
    boxes_t = jnp.transpose(gt_boxes, (0, 2, 1))
    lab_f = gt_labels.astype(jnp.float32).reshape(B, 1, n_obj)

    grid = (B, nblocks)
    ca_spec = pl.BlockSpec((4, BN), lambda b, k: (0, k))
    box_spec = pl.BlockSpec((1, n_obj, 4), lambda b, k: (b, 0, 0))
    bt_spec = pl.BlockSpec((1, 4, n_obj), lambda b, k: (b, 0, 0))
    row_spec = pl.BlockSpec((1, 1, BN), lambda b, k: (b, 0, k))
    po_spec_a = pl.BlockSpec((1, 1, n_obj, 1), lambda b, k: (b, k, 0, 0))
    po_spec_b = pl.BlockSpec((1, nblocks, n_obj, 1), lambda b, k: (b, 0, 0, 0))

    m, am, pom, poa = pl.pallas_call(
        functools.partial(_pass_a, BN),
        grid=grid,
        in_specs=[ca_spec, box_spec],
        out_specs=[row_spec, row_spec, po_spec_a, po_spec_a],
        out_shape=[
            jax.ShapeDtypeStruct((B, 1, np_pad), jnp.float32),
            jax.ShapeDtypeStruct((B, 1, np_pad), jnp.int32),
            jax.ShapeDtypeStruct((B, nblocks, n_obj, 1), jnp.float32),
            jax.ShapeDtypeStruct((B, nblocks, n_obj, 1), jnp.int32),
        ],
    )(ca_t, gt_boxes)

    cls, loc, aid = pl.pallas_call(
        functools.partial(_pass_b, BN, nblocks),
        grid=grid,
        in_specs=[ca_spec, bt_spec,
                  pl.BlockSpec((1, 1, n_obj), lambda b, k: (b, 0, 0)),
                  row_spec, row_spec, po_spec_b, po_spec_b],
        out_specs=[
            pl.BlockSpec((1, BN, 128), lambda b, k: (b, k, 0)),
            pl.BlockSpec((1, 4, BN), lambda b, k: (b, 0, k)),
            pl.BlockSpec((1, 1, BN), lambda b, k: (b, 0, k)),
        ],
        out_shape=[
            jax.ShapeDtypeStruct((B, np_real, 128), jnp.float32),
            jax.ShapeDtypeStruct((B, 4, np_real), jnp.float32),
            jax.ShapeDtypeStruct((B, 1, np_real), jnp.float32),
        ],
    )(ca_t, boxes_t, lab_f, m, am, pom, poa)

    return (cls[:, :, :NC], jnp.transpose(loc, (0, 2, 1)),
            aid.reshape(B, np_real))


# speedup vs baseline: 1.0125x; 1.0125x over previous
import functools

import jax
import jax.numpy as jnp
from jax.experimental import pallas as pl

NC = 81
BIGF = 3.0e38
BIGI = 2 ** 30
HIGHEST = jax.lax.Precision.HIGHEST


def _pass_a(bn, ca_ref, box_ref, m_ref, am_ref, pom_ref, poa_ref):
    kb = pl.program_id(1)
    ca = ca_ref[...]
    ax1 = ca[0:1, :] - ca[2:3, :] / 2.0
    ay1 = ca[1:2, :] - ca[3:4, :] / 2.0
    ax2 = ca[0:1, :] + ca[2:3, :] / 2.0
    ay2 = ca[1:2, :] + ca[3:4, :] / 2.0
    b = box_ref[0]
    bx1, by1, bx2, by2 = b[:, 0:1], b[:, 1:2], b[:, 2:3], b[:, 3:4]
    iw = jnp.clip(jnp.minimum(ax2, bx2) - jnp.maximum(ax1, bx1), 0.0)
    ih = jnp.clip(jnp.minimum(ay2, by2) - jnp.maximum(ay1, by1), 0.0)
    inter = iw * ih
    aarea = (ax2 - ax1) * (ay2 - ay1)
    barea = (bx2 - bx1) * (by2 - by1)
    iou = inter / (aarea + barea - inter)

    m = jnp.max(iou, axis=0, keepdims=True)
    iids = jax.lax.broadcasted_iota(jnp.int32, iou.shape, 0)
    am = jnp.min(jnp.where(iou == m, iids, BIGI), axis=0, keepdims=True)
    m_ref[...] = m.reshape(1, 1, bn)
    am_ref[...] = am.reshape(1, 1, bn)

    pm = jnp.max(iou, axis=1, keepdims=True)
    gids = jax.lax.broadcasted_iota(jnp.int32, iou.shape, 1) + kb * bn
    pa = jnp.min(jnp.where(iou == pm, gids, BIGI), axis=1, keepdims=True)
    pom_ref[...] = pm.reshape(1, 1, -1, 1)
    poa_ref[...] = pa.reshape(1, 1, -1, 1)


def _pass_b(bn, nblocks, ca_ref, bt_ref, lab_ref, m_ref, am_ref, pom_ref,
            poa_ref, cls_ref, loc_ref, aid_ref):
    kb = pl.program_id(1)
    n_obj = bt_ref.shape[-1]

    best = pom_ref[0, 0]
    for k2 in range(1, nblocks):
        best = jnp.maximum(best, pom_ref[0, k2])
    gidx = jnp.full_like(poa_ref[0, 0], BIGI)
    for k2 in range(nblocks):
        gidx = jnp.minimum(
            gidx, jnp.where(pom_ref[0, k2] == best, poa_ref[0, k2], BIGI))
    any_pos = jnp.max(best) >= 0.5

    m = m_ref[0]
    am = am_ref[0]
    sel = (jax.lax.broadcasted_iota(jnp.int32, (n_obj, bn), 0)
           == am).astype(jnp.float32)

    bt = bt_ref[0]
    lab5 = jnp.concatenate([bt, lab_ref[0]], axis=0)
    bl = jax.lax.dot_general(lab5, sel, (((1,), (0,)), ((), ())),
                             precision=HIGHEST)

    gi = jax.lax.broadcasted_iota(jnp.int32, (n_obj, bn), 1) + kb * bn
    fb = jnp.any(gi == gidx, axis=0, keepdims=True)
    pos = (any_pos & (m >= 0.5)) | (jnp.logical_not(any_pos) & fb)
    neg = m < 0.4
    posf = pos.astype(jnp.float32)
    ignf = (1.0 - posf) * (1.0 - neg.astype(jnp.float32))
    aid_ref[...] = (posf - ignf).reshape(1, 1, bn)

    onehot = (jax.lax.broadcasted_iota(jnp.int32, (n_obj, 128), 1)
              == lab_ref[0].astype(jnp.int32).reshape(n_obj, 1) + 1
              ).astype(jnp.float32)
    ign_row = jnp.where(
        jax.lax.broadcasted_iota(jnp.int32, (1, 128), 1) < NC, -1.0, 0.0)
    mmat = jnp.concatenate([onehot, ign_row], axis=0)
    smat = jnp.concatenate([sel * posf, ignf], axis=0)
    cls = jax.lax.dot_general(smat, mmat, (((0,), (0,)), ((), ())),
                              precision=jax.lax.Precision.DEFAULT)
    cls_ref[...] = cls.reshape(1, bn, 128)

    ca = ca_ref[...]
    x1, y1, x2, y2 = bl[0:1, :], bl[1:2, :], bl[2:3, :], bl[3:4, :]
    cx = (x1 + x2) / 2.0
    cy = (y1 + y2) / 2.0
    w = x2 - x1
    h = y2 - y1
    gcx = (cx - ca[0:1, :]) / (ca[2:3, :] / 10.0)
    gcy = (cy - ca[1:2, :]) / (ca[3:4, :] / 10.0)
    gw = jnp.log(w / ca[2:3, :]) * 5.0
    gh = jnp.log(h / ca[3:4, :]) * 5.0
    loc_t = jnp.concatenate([gcx, gcy, gw, gh], axis=0)
    loc_ref[...] = loc_t.reshape(1, 4, bn)


@jax.jit
def kernel(gt_boxes, gt_labels, center_anchor):
    B, n_obj = gt_labels.shape
    np_real = center_anchor.shape[0]
    BN = 6784
    nblocks = -(-np_real // BN)
    np_pad = nblocks * BN

    pad = jnp.broadcast_to(jnp.array([2.0, 2.0, 0.01, 0.01], jnp.float32),
                           (np_pad - np_real, 4))
    ca_t = jnp.transpose(jnp.concatenate([center_anchor, pad], axis=0))
    boxes_t = jnp.transpose(gt_boxes, (0, 2, 1))
    lab_f = gt_labels.astype(jnp.float32).reshape(B, 1, n_obj)

    grid = (B, nblocks)
    ca_spec = pl.BlockSpec((4, BN), lambda b, k: (0, k))
    box_spec = pl.BlockSpec((1, n_obj, 4), lambda b, k: (b, 0, 0))
    bt_spec = pl.BlockSpec((1, 4, n_obj), lambda b, k: (b, 0, 0))
    row_spec = pl.BlockSpec((1, 1, BN), lambda b, k: (b, 0, k))
    po_spec_a = pl.BlockSpec((1, 1, n_obj, 1), lambda b, k: (b, k, 0, 0))
    po_spec_b = pl.BlockSpec((1, nblocks, n_obj, 1), lambda b, k: (b, 0, 0, 0))

    m, am, pom, poa = pl.pallas_call(
        functools.partial(_pass_a, BN),
        grid=grid,
        in_specs=[ca_spec, box_spec],
        out_specs=[row_spec, row_spec, po_spec_a, po_spec_a],
        out_shape=[
            jax.ShapeDtypeStruct((B, 1, np_pad), jnp.float32),
            jax.ShapeDtypeStruct((B, 1, np_pad), jnp.int32),
            jax.ShapeDtypeStruct((B, nblocks, n_obj, 1), jnp.float32),
            jax.ShapeDtypeStruct((B, nblocks, n_obj, 1), jnp.int32),
        ],
    )(ca_t, gt_boxes)

    cls, loc, aid = pl.pallas_call(
        functools.partial(_pass_b, BN, nblocks),
        grid=grid,
        in_specs=[ca_spec, bt_spec,
                  pl.BlockSpec((1, 1, n_obj), lambda b, k: (b, 0, 0)),
                  row_spec, row_spec, po_spec_b, po_spec_b],
        out_specs=[
            pl.BlockSpec((1, BN, 128), lambda b, k: (b, k, 0)),
            pl.BlockSpec((1, 4, BN), lambda b, k: (b, 0, k)),
            pl.BlockSpec((1, 1, BN), lambda b, k: (b, 0, k)),
        ],
        out_shape=[
            jax.ShapeDtypeStruct((B, np_real, 128), jnp.float32),
            jax.ShapeDtypeStruct((B, 4, np_real), jnp.float32),
            jax.ShapeDtypeStruct((B, 1, np_real), jnp.float32),
        ],
    )(ca_t, boxes_t, lab_f, m, am, pom, poa)

    return (cls[:, :, :NC], jnp.transpose(loc, (0, 2, 1)),
            aid.reshape(B, np_real))
